# baseline (speedup 1.0000x reference)
import functools

import jax
import jax.numpy as jnp
from jax import lax
from jax.experimental import pallas as pl
from jax.experimental.pallas import tpu as pltpu
from jax.experimental.pallas import tpu_sc as plsc

N_SEL = 8192
D = 256
L = 16
NT = 16
CHUNK = N_SEL // NT
IB = 128
OUTW = D // NT
PW = D + L


def _sc_kernel(triple_heads, triple_indices, cls_embeddings, tgt_vec,
               entity_emb, W, b):
    mesh = plsc.VectorSubcoreMesh(
        core_axis_name="c", subcore_axis_name="s", num_cores=1)

    @functools.partial(
        pl.kernel,
        out_type=[
            jax.ShapeDtypeStruct((D,), jnp.float32),
            jax.ShapeDtypeStruct((NT, PW), jnp.float32),
        ],
        mesh=mesh,
        scratch_types=[
            pltpu.VMEM((CHUNK // IB, IB), jnp.int32),
            pltpu.VMEM((CHUNK // IB, IB), jnp.int32),
            pltpu.VMEM((CHUNK,), jnp.float32),
            pltpu.VMEM((L, D), jnp.float32),
            pltpu.VMEM((PW,), jnp.float32),
            pltpu.VMEM((L,), jnp.int32),
            pltpu.VMEM((1, D), jnp.float32),
            pltpu.VMEM((NT, PW), jnp.float32),
            pltpu.VMEM((OUTW, D), jnp.float32),
            pltpu.VMEM((L,), jnp.float32),
            pltpu.VMEM((L,), jnp.float32),
            pltpu.VMEM((D,), jnp.float32),
            pltpu.SemaphoreType.DMA,
        ],
    )
    def k(th_hbm, ti_hbm, cls_hbm, tgt_hbm, ent_hbm, w_hbm, b_hbm,
          out_hbm, stage_hbm,
          idx_v, heads_v, wf_v, rows_v, acc_v, tgt_v, entrow_v,
          part_v, wrow_v, b_v, out_v, mean_v, sem):
        wid = lax.axis_index("s")
        base = wid * CHUNK

        pltpu.sync_copy(tgt_hbm, tgt_v)
        tgtv = tgt_v[...]

        for j in range(CHUNK // IB):
            pltpu.sync_copy(ti_hbm.at[pl.ds(base + j * IB, IB)], idx_v.at[j])
        for j in range(CHUNK // IB):
            pltpu.async_copy(th_hbm.at[idx_v.at[j]], heads_v.at[j], sem).wait()

        zf = jnp.zeros((L,), jnp.float32)
        for j in range(PW // L):
            acc_v[pl.ds(j * L, L)] = zf

        cw = jnp.zeros((L,), jnp.float32)
        for ci in range(CHUNK // L):
            hv = heads_v[ci // (IB // L), pl.ds((ci % (IB // L)) * L, L)]
            m = hv == tgtv
            wf = jnp.where(m, 1.0, 0.0)
            wf_v[pl.ds(ci * L, L)] = wf
            cw = cw + wf
        acc_v[pl.ds(D, L)] = cw

        def bbody(ci, carry):
            wv = wf_v[pl.ds(ci * L, L)]
            ws = [wv[r] for r in range(L)]
            msum = ws[0]
            for r in range(1, L):
                msum = msum + ws[r]

            @pl.when(msum > 0.0)
            def _():
                pltpu.sync_copy(cls_hbm.at[pl.ds(base + ci * L, L)], rows_v)
                for dc in range(D // L):
                    a = acc_v[pl.ds(dc * L, L)]
                    for r in range(L):
                        a = a + rows_v[r, pl.ds(dc * L, L)] * ws[r]
                    acc_v[pl.ds(dc * L, L)] = a

            return carry

        lax.fori_loop(0, CHUNK // L, bbody, jnp.int32(0))

        pltpu.sync_copy(acc_v, stage_hbm.at[wid])
        plsc.subcore_barrier()
        pltpu.sync_copy(stage_hbm, part_v)

        cv = part_v[0, pl.ds(D, L)]
        for t in range(1, NT):
            cv = cv + part_v[t, pl.ds(D, L)]
        cnt_f = cv[0]
        for rr in range(1, L):
            cnt_f = cnt_f + cv[rr]

        x_s = jnp.maximum(cnt_f, 1.0)
        xi_s = lax.bitcast_convert_type(x_s, jnp.int32)
        r_s = lax.bitcast_convert_type(0x7EF311C3 - xi_s, jnp.float32)
        for _ in range(3):
            r_s = r_s * (2.0 - x_s * r_s)
        r = jnp.broadcast_to(r_s, (L,))

        for dc in range(D // L):
            a = part_v[0, pl.ds(dc * L, L)]
            for t in range(1, NT):
                a = a + part_v[t, pl.ds(dc * L, L)]
            mean_v[pl.ds(dc * L, L)] = a * r

        obase = wid * OUTW
        pltpu.sync_copy(w_hbm.at[pl.ds(obase, OUTW)], wrow_v)
        pltpu.sync_copy(b_hbm.at[pl.ds(obase, L)], b_v)
        t_s = tgtv[0]
        pltpu.sync_copy(ent_hbm.at[pl.ds(t_s, 1)], entrow_v)

        outs = []
        for j in range(OUTW):
            dv = jnp.zeros((L,), jnp.float32)
            for dc in range(D // L):
                dv = dv + wrow_v[j, pl.ds(dc * L, L)] * mean_v[pl.ds(dc * L, L)]
            ssum = dv[0]
            for rr in range(1, L):
                ssum = ssum + dv[rr]
            outs.append(ssum)

        lanes = lax.iota(jnp.int32, L)
        ov = jnp.zeros((L,), jnp.float32)
        for j in range(OUTW):
            ov = jnp.where(lanes == j, outs[j], ov)
        ov = ov + b_v[...]

        sel = jnp.where(cnt_f > 0.0, 1.0, 0.0)
        ev = entrow_v[0, pl.ds(obase, L)]
        out_v[...] = ov * sel + ev * (1.0 - sel)
        pltpu.sync_copy(out_v, out_hbm.at[pl.ds(obase, L)])

    out, _ = k(triple_heads, triple_indices, cls_embeddings, tgt_vec,
               entity_emb, W, b)
    return out


def kernel(entity_emb, cls_embeddings, triple_heads, triple_indices, target_head_id, W, b):
    tgt_vec = jnp.broadcast_to(jnp.asarray(target_head_id, jnp.int32), (L,))
    return _sc_kernel(triple_heads, triple_indices, cls_embeddings, tgt_vec,
                      entity_emb, W, b)


# speedup vs baseline: 1.2796x; 1.2796x over previous
import functools

import jax
import jax.numpy as jnp
from jax import lax
from jax.experimental import pallas as pl
from jax.experimental.pallas import tpu as pltpu
from jax.experimental.pallas import tpu_sc as plsc

N_SEL = 8192
D = 256
L = 16
NC, NS = 2, 16
NW = NC * NS
CHUNK = N_SEL // NW
IB = 128
PW = D + L


def _sc_stage(triple_heads, triple_indices, cls_embeddings, tgt_vec,
              entity_emb):
    mesh = plsc.VectorSubcoreMesh(core_axis_name="c", subcore_axis_name="s")

    @functools.partial(
        pl.kernel,
        out_type=[
            jax.ShapeDtypeStruct((NW, PW), jnp.float32),
            jax.ShapeDtypeStruct((1, D), jnp.float32),
        ],
        mesh=mesh,
        scratch_types=[
            pltpu.VMEM((CHUNK // IB, IB), jnp.int32),
            pltpu.VMEM((CHUNK,), jnp.int32),
            pltpu.VMEM((L, D), jnp.float32),
            pltpu.VMEM((PW,), jnp.float32),
            pltpu.VMEM((L,), jnp.int32),
            pltpu.VMEM((1, D), jnp.float32),
            pltpu.SemaphoreType.DMA,
            pltpu.SemaphoreType.DMA,
        ],
    )
    def k(th_hbm, ti_hbm, cls_hbm, tgt_hbm, ent_hbm,
          part_out, ent_out,
          idx_v, heads_v, rows_v, acc_v, tgt_v, entrow_v, sem, sem2):
        c = lax.axis_index("c")
        s = lax.axis_index("s")
        wid = s * NC + c
        base = wid * CHUNK

        pltpu.sync_copy(tgt_hbm, tgt_v)
        tgtv = tgt_v[...]
        t_s = tgtv[0]

        cps = [pltpu.async_copy(ti_hbm.at[pl.ds(base + j * IB, IB)],
                                idx_v.at[j], sem)
               for j in range(CHUNK // IB)]
        for cp in cps:
            cp.wait()
        cps = [pltpu.async_copy(th_hbm.at[idx_v.at[j]],
                                heads_v.at[pl.ds(j * IB, IB)], sem2)
               for j in range(CHUNK // IB)]
        for cp in cps:
            cp.wait()

        zf = jnp.zeros((L,), jnp.float32)
        for j in range(D // L):
            acc_v[pl.ds(j * L, L)] = zf

        def bbody(ci, cw):
            hv = heads_v[pl.ds(ci * L, L)]
            m = hv == tgtv
            wf = jnp.where(m, 1.0, 0.0)
            ws = [wf[r] for r in range(L)]
            msum = ws[0]
            for r in range(1, L):
                msum = msum + ws[r]

            @pl.when(msum > 0.0)
            def _():
                pltpu.sync_copy(cls_hbm.at[pl.ds(base + ci * L, L)], rows_v)
                for dc in range(D // L):
                    a = acc_v[pl.ds(dc * L, L)]
                    for r in range(L):
                        a = a + rows_v[r, pl.ds(dc * L, L)] * ws[r]
                    acc_v[pl.ds(dc * L, L)] = a

            return cw + wf

        cw = lax.fori_loop(0, CHUNK // L, bbody, jnp.zeros((L,), jnp.float32))
        acc_v[pl.ds(D, L)] = cw

        pltpu.sync_copy(acc_v, part_out.at[wid])

        @pl.when(wid == 0)
        def _():
            pltpu.sync_copy(ent_hbm.at[pl.ds(t_s, 1)], entrow_v)
            pltpu.sync_copy(entrow_v, ent_out)

    return k(triple_heads, triple_indices, cls_embeddings, tgt_vec,
             entity_emb)


def _tc_stage(part, entrow, W, b):
    def body(part_ref, ent_ref, w_ref, b_ref, out_ref):
        summed = jnp.sum(part_ref[:, :D], axis=0)
        cnt_f = jnp.sum(part_ref[:, D:])
        mean = summed / jnp.maximum(cnt_f, 1.0)
        upd = lax.dot_general(mean[None, :], w_ref[...],
                              (((1,), (1,)), ((), ())),
                              preferred_element_type=jnp.float32)[0] + b_ref[...]
        out_ref[...] = jnp.where(cnt_f > 0.0, upd, ent_ref[0])

    return pl.pallas_call(
        body,
        out_shape=jax.ShapeDtypeStruct((D,), jnp.float32),
    )(part, entrow, W, b)


def kernel(entity_emb, cls_embeddings, triple_heads, triple_indices, target_head_id, W, b):
    tgt_vec = jnp.broadcast_to(jnp.asarray(target_head_id, jnp.int32), (L,))
    part, entrow = _sc_stage(
        triple_heads, triple_indices, cls_embeddings, tgt_vec, entity_emb)
    return _tc_stage(part, entrow, W, b)


# speedup vs baseline: 1.2907x; 1.0087x over previous
import functools

import jax
import jax.numpy as jnp
from jax import lax
from jax.experimental import pallas as pl
from jax.experimental.pallas import tpu as pltpu
from jax.experimental.pallas import tpu_sc as plsc

N_SEL = 8192
D = 256
L = 16
NC, NS = 2, 16
NW = NC * NS
CHUNK = N_SEL // NW
IB = 128
PW = D + L


def _sc_stage(triple_heads, triple_indices, cls_embeddings, tgt_vec,
              entity_emb):
    mesh = plsc.VectorSubcoreMesh(core_axis_name="c", subcore_axis_name="s")

    @functools.partial(
        pl.kernel,
        out_type=[
            jax.ShapeDtypeStruct((NW, PW), jnp.float32),
            jax.ShapeDtypeStruct((1, D), jnp.float32),
        ],
        mesh=mesh,
        scratch_types=[
            pltpu.VMEM((CHUNK // IB, IB), jnp.int32),
            pltpu.VMEM((CHUNK,), jnp.int32),
            pltpu.VMEM((L, D), jnp.float32),
            pltpu.VMEM((PW,), jnp.float32),
            pltpu.VMEM((L,), jnp.int32),
            pltpu.VMEM((1, D), jnp.float32),
            pltpu.SemaphoreType.DMA,
            pltpu.SemaphoreType.DMA,
        ],
    )
    def k(th_hbm, ti_hbm, cls_hbm, tgt_hbm, ent_hbm,
          part_out, ent_out,
          idx_v, heads_v, rows_v, acc_v, tgt_v, entrow_v, sem, sem2):
        c = lax.axis_index("c")
        s = lax.axis_index("s")
        wid = s * NC + c
        base = wid * CHUNK

        tcp = pltpu.async_copy(tgt_hbm, tgt_v, sem)
        cps = [pltpu.async_copy(ti_hbm.at[pl.ds(base + j * IB, IB)],
                                idx_v.at[j], sem2)
               for j in range(CHUNK // IB)]
        tcp.wait()
        tgtv = tgt_v[...]
        t_s = tgtv[0]
        ecp = pltpu.async_copy(ent_hbm.at[pl.ds(t_s, 1)], entrow_v, sem)
        for cp in cps:
            cp.wait()
        cps = [pltpu.async_copy(th_hbm.at[idx_v.at[j]],
                                heads_v.at[pl.ds(j * IB, IB)], sem2)
               for j in range(CHUNK // IB)]
        for cp in cps:
            cp.wait()

        zf = jnp.zeros((L,), jnp.float32)
        for j in range(D // L):
            acc_v[pl.ds(j * L, L)] = zf

        cw = jnp.zeros((L,), jnp.float32)
        wfs = []
        for ci in range(CHUNK // L):
            hv = heads_v[pl.ds(ci * L, L)]
            wf = jnp.where(hv == tgtv, 1.0, 0.0)
            wfs.append(wf)
            cw = cw + wf
        acc_v[pl.ds(D, L)] = cw
        tile_total = cw[0]
        for r in range(1, L):
            tile_total = tile_total + cw[r]

        @pl.when(tile_total > 0.0)
        def _():
            def bbody(ci, carry):
                hv = heads_v[pl.ds(ci * L, L)]
                wf = jnp.where(hv == tgtv, 1.0, 0.0)
                ws = [wf[r] for r in range(L)]
                msum = ws[0]
                for r in range(1, L):
                    msum = msum + ws[r]

                @pl.when(msum > 0.0)
                def _():
                    pltpu.sync_copy(cls_hbm.at[pl.ds(base + ci * L, L)],
                                    rows_v)
                    for dc in range(D // L):
                        a = acc_v[pl.ds(dc * L, L)]
                        for r in range(L):
                            a = a + rows_v[r, pl.ds(dc * L, L)] * ws[r]
                        acc_v[pl.ds(dc * L, L)] = a

                return carry

            lax.fori_loop(0, CHUNK // L, bbody, jnp.int32(0))

        pltpu.sync_copy(acc_v, part_out.at[wid])
        ecp.wait()

        @pl.when(wid == 0)
        def _():
            pltpu.sync_copy(entrow_v, ent_out)

    return k(triple_heads, triple_indices, cls_embeddings, tgt_vec,
             entity_emb)


def _tc_stage(part, entrow, W, b):
    def body(part_ref, ent_ref, w_ref, b_ref, out_ref):
        summed = jnp.sum(part_ref[:, :D], axis=0)
        cnt_f = jnp.sum(part_ref[:, D:])
        mean = summed / jnp.maximum(cnt_f, 1.0)
        upd = lax.dot_general(mean[None, :], w_ref[...],
                              (((1,), (1,)), ((), ())),
                              preferred_element_type=jnp.float32)[0] + b_ref[...]
        out_ref[...] = jnp.where(cnt_f > 0.0, upd, ent_ref[0])

    return pl.pallas_call(
        body,
        out_shape=jax.ShapeDtypeStruct((D,), jnp.float32),
    )(part, entrow, W, b)


def kernel(entity_emb, cls_embeddings, triple_heads, triple_indices, target_head_id, W, b):
    tgt_vec = jnp.broadcast_to(jnp.asarray(target_head_id, jnp.int32), (L,))
    part, entrow = _sc_stage(
        triple_heads, triple_indices, cls_embeddings, tgt_vec, entity_emb)
    return _tc_stage(part, entrow, W, b)
